# baseline (speedup 1.0000x reference)
import jax
import jax.numpy as jnp
from jax.experimental import pallas as pl

IDIM = 512
EMBED = 1000
TB = 512


def _body(x_ref, w_ref, out_ref, idx_ref):
    x = x_ref[...]
    w = w_ref[...]
    inv_norm = jax.lax.rsqrt(jnp.sum(w * w, axis=1))
    sim = jax.lax.dot_general(x, w, (((1,), (1,)), ((), ())),
                              preferred_element_type=jnp.float32)
    sim = sim * inv_norm[None, :]
    m = jnp.max(sim, axis=1, keepdims=True)
    eids = jax.lax.broadcasted_iota(jnp.int32, sim.shape, 1)
    idx = jnp.min(jnp.where(sim == m, eids, EMBED), axis=1)
    oh = (eids == idx[:, None]).astype(jnp.bfloat16)
    anchor = jax.lax.dot_general(oh, w.astype(jnp.bfloat16),
                                 (((1,), (0,)), ((), ())),
                                 preferred_element_type=jnp.float32)
    a = anchor * x
    am = jnp.max(a, axis=1, keepdims=True)
    e = jnp.exp(a - am)
    g = e / jnp.sum(e, axis=1, keepdims=True)
    out_ref[...] = g * anchor
    idx_ref[0, 0, :] = idx


def kernel(xs_pad_in, embed_weight):
    B, T, D = xs_pad_in.shape
    N = B * T
    nb = N // TB
    x2 = xs_pad_in.reshape(N, D)
    out, idx = pl.pallas_call(
        _body,
        grid=(nb,),
        in_specs=[pl.BlockSpec((TB, D), lambda i: (i, 0)),
                  pl.BlockSpec((EMBED, D), lambda i: (0, 0))],
        out_specs=[pl.BlockSpec((TB, D), lambda i: (i, 0)),
                   pl.BlockSpec((1, 1, TB), lambda i: (i, 0, 0))],
        out_shape=[jax.ShapeDtypeStruct((N, D), jnp.float32),
                   jax.ShapeDtypeStruct((nb, 1, TB), jnp.int32)],
    )(x2, embed_weight)
    anchors = out.reshape(B, 1, T, D)
    score_idxs = idx.reshape(B, 1, T)
    return anchors, score_idxs


# speedup vs baseline: 1.0099x; 1.0099x over previous
import jax
import jax.numpy as jnp
from jax.experimental import pallas as pl
from jax.experimental.pallas import tpu as pltpu

IDIM = 512
EMBED = 1000
TB = 512
EPAD = 1024


def _body(x_ref, w_ref, out_ref, idx_ref, inv_ref, wb_ref):
    @pl.when(pl.program_id(0) == 0)
    def _():
        w = w_ref[...]
        inv_ref[...] = jax.lax.rsqrt(jnp.sum(w * w, axis=1))[None, :]
        wpad = jnp.concatenate(
            [w, jnp.zeros((EPAD - EMBED, IDIM), jnp.float32)], axis=0)
        wb_ref[...] = wpad.astype(jnp.bfloat16)

    x = x_ref[...]
    sim = jax.lax.dot_general(x, w_ref[...], (((1,), (1,)), ((), ())),
                              preferred_element_type=jnp.float32)
    sim = sim * inv_ref[...]
    m = jnp.max(sim, axis=1, keepdims=True)
    eids = jax.lax.broadcasted_iota(jnp.int32, sim.shape, 1)
    idx = jnp.min(jnp.where(sim == m, eids, EMBED), axis=1)
    eids_pad = jax.lax.broadcasted_iota(jnp.int32, (TB, EPAD), 1)
    oh = (eids_pad == idx[:, None]).astype(jnp.bfloat16)
    anchor = jax.lax.dot_general(oh, wb_ref[...], (((1,), (0,)), ((), ())),
                                 preferred_element_type=jnp.float32)
    a = anchor * x
    am = jnp.max(a, axis=1, keepdims=True)
    e = jnp.exp(a - am)
    g = e / jnp.sum(e, axis=1, keepdims=True)
    out_ref[...] = g * anchor
    idx_ref[0, 0, :] = idx


def kernel(xs_pad_in, embed_weight):
    B, T, D = xs_pad_in.shape
    N = B * T
    nb = N // TB
    x2 = xs_pad_in.reshape(N, D)
    out, idx = pl.pallas_call(
        _body,
        grid=(nb,),
        in_specs=[pl.BlockSpec((TB, D), lambda i: (i, 0)),
                  pl.BlockSpec((EMBED, D), lambda i: (0, 0))],
        out_specs=[pl.BlockSpec((TB, D), lambda i: (i, 0)),
                   pl.BlockSpec((1, 1, TB), lambda i: (i, 0, 0))],
        out_shape=[jax.ShapeDtypeStruct((N, D), jnp.float32),
                   jax.ShapeDtypeStruct((nb, 1, TB), jnp.int32)],
        scratch_shapes=[pltpu.VMEM((1, EMBED), jnp.float32),
                        pltpu.VMEM((EPAD, IDIM), jnp.bfloat16)],
    )(x2, embed_weight)
    anchors = out.reshape(B, 1, T, D)
    score_idxs = idx.reshape(B, 1, T)
    return anchors, score_idxs


# speedup vs baseline: 1.1557x; 1.1443x over previous
import jax
import jax.numpy as jnp
from jax.experimental import pallas as pl
from jax.experimental.pallas import tpu as pltpu

IDIM = 512
EMBED = 1000
TB = 1024
EPAD = 1024


def _body(x_ref, w_ref, out_ref, idx_ref, inv_ref, wb_ref):
    @pl.when(pl.program_id(0) == 0)
    def _():
        w = w_ref[...]
        inv_ref[...] = jax.lax.rsqrt(jnp.sum(w * w, axis=1))[None, :]
        wpad = jnp.concatenate(
            [w, jnp.zeros((EPAD - EMBED, IDIM), jnp.float32)], axis=0)
        wb_ref[...] = wpad.astype(jnp.bfloat16)

    x = x_ref[...]
    sim = jax.lax.dot_general(x, w_ref[...], (((1,), (1,)), ((), ())),
                              preferred_element_type=jnp.float32)
    sim = sim * inv_ref[...]
    m = jnp.max(sim, axis=1, keepdims=True)
    eids = jax.lax.broadcasted_iota(jnp.int32, sim.shape, 1)
    idx = jnp.min(jnp.where(sim == m, eids, EMBED), axis=1)
    eids_pad = jax.lax.broadcasted_iota(jnp.int32, (TB, EPAD), 1)
    oh = (eids_pad == idx[:, None]).astype(jnp.bfloat16)
    anchor = jax.lax.dot_general(oh, wb_ref[...], (((1,), (0,)), ((), ())),
                                 preferred_element_type=jnp.float32)
    a = anchor * x
    am = jnp.max(a, axis=1, keepdims=True)
    e = jnp.exp(a - am)
    g = e / jnp.sum(e, axis=1, keepdims=True)
    out_ref[...] = g * anchor
    idx_ref[0, 0, :] = idx


def kernel(xs_pad_in, embed_weight):
    B, T, D = xs_pad_in.shape
    N = B * T
    nb = N // TB
    x2 = xs_pad_in.reshape(N, D)
    out, idx = pl.pallas_call(
        _body,
        grid=(nb,),
        in_specs=[pl.BlockSpec((TB, D), lambda i: (i, 0)),
                  pl.BlockSpec((EMBED, D), lambda i: (0, 0))],
        out_specs=[pl.BlockSpec((TB, D), lambda i: (i, 0)),
                   pl.BlockSpec((1, 1, TB), lambda i: (i, 0, 0))],
        out_shape=[jax.ShapeDtypeStruct((N, D), jnp.float32),
                   jax.ShapeDtypeStruct((nb, 1, TB), jnp.int32)],
        scratch_shapes=[pltpu.VMEM((1, EMBED), jnp.float32),
                        pltpu.VMEM((EPAD, IDIM), jnp.bfloat16)],
    )(x2, embed_weight)
    anchors = out.reshape(B, 1, T, D)
    score_idxs = idx.reshape(B, 1, T)
    return anchors, score_idxs
